# baseline (speedup 1.0000x reference)
import jax
import jax.numpy as jnp
from jax import lax
from jax.experimental import pallas as pl
from jax.experimental.pallas import tpu as pltpu
from jax.experimental.pallas import tpu_sc as plsc

B = 16384
EMB = 32
K = 4
KD = EMB * K

NC = 2
NS = 16
NW = NC * NS
BPW = B // NW
CH = 128


def _sc_gather_body(uid_hbm, iid_hbm, uemb_hbm, iemb_hbm, ubias_hbm, ibias_hbm,
                    ue_out, ie_out, ub_out, ib_out,
                    ue_v, ie_v, ub_v, ib_v, uid_s, iid_s, sem):
    wid = lax.axis_index("s") * NC + lax.axis_index("c")
    base = wid * BPW
    pltpu.sync_copy(uid_hbm.at[pl.ds(base, BPW)], uid_s)
    pltpu.sync_copy(iid_hbm.at[pl.ds(base, BPW)], iid_s)

    def chunk(c, _):
        cbase = c * CH
        copies = []
        for g in range(CH // 16):
            uv = uid_s[pl.ds(cbase + g * 16, 16)]
            iv = iid_s[pl.ds(cbase + g * 16, 16)]
            for t in range(16):
                u = g * 16 + t
                ur = uv[t]
                ir = iv[t]
                copies.append(pltpu.make_async_copy(
                    uemb_hbm.at[pl.ds(ur, 1)], ue_v.at[pl.ds(u, 1)], sem))
                copies.append(pltpu.make_async_copy(
                    iemb_hbm.at[pl.ds(ir, 1)], ie_v.at[pl.ds(u, 1)], sem))
                copies.append(pltpu.make_async_copy(
                    ubias_hbm.at[pl.ds(ur, 1)], ub_v.at[pl.ds(u, 1)], sem))
                copies.append(pltpu.make_async_copy(
                    ibias_hbm.at[pl.ds(ir, 1)], ib_v.at[pl.ds(u, 1)], sem))
        for cp in copies:
            cp.start()
        for cp in copies:
            cp.wait()
        pltpu.sync_copy(ue_v, ue_out.at[pl.ds(base + cbase, CH)])
        pltpu.sync_copy(ie_v, ie_out.at[pl.ds(base + cbase, CH)])
        pltpu.sync_copy(ub_v, ub_out.at[pl.ds(base + cbase, CH)])
        pltpu.sync_copy(ib_v, ib_out.at[pl.ds(base + cbase, CH)])

    lax.fori_loop(0, BPW // CH, chunk, None)


def _sc_gather(uids, iids, uemb, iemb, ubias, ibias):
    mesh = plsc.VectorSubcoreMesh(core_axis_name="c", subcore_axis_name="s",
                                  num_cores=NC, num_subcores=NS)
    f = pl.kernel(
        _sc_gather_body,
        out_type=(
            jax.ShapeDtypeStruct((B, EMB), jnp.float32),
            jax.ShapeDtypeStruct((B, EMB), jnp.float32),
            jax.ShapeDtypeStruct((B, 1), jnp.float32),
            jax.ShapeDtypeStruct((B, 1), jnp.float32),
        ),
        mesh=mesh,
        compiler_params=pltpu.CompilerParams(use_tc_tiling_on_sc=True),
        scratch_types=[
            pltpu.VMEM((CH, EMB), jnp.float32),
            pltpu.VMEM((CH, EMB), jnp.float32),
            pltpu.VMEM((CH, 1), jnp.float32),
            pltpu.VMEM((CH, 1), jnp.float32),
            pltpu.VMEM((BPW,), jnp.int32),
            pltpu.VMEM((BPW,), jnp.int32),
            pltpu.SemaphoreType.DMA,
        ],
    )
    return f(uids, iids, uemb, iemb, ubias, ibias)


N4 = B // 4
BLKN = 512
W4 = 4 * KD


def _tc_mix_body(pue_ref, pie_ref, pub_ref, pib_ref,
                 wt_ref, bt_ref, wa_ref, ba_ref, out_ref):
    p = pue_ref[...]
    g = pie_ref[...]
    wt = wt_ref[...]
    wa = wa_ref[...]
    mi = lax.broadcasted_iota(jnp.int32, (KD, W4), 0) // EMB
    mj = lax.broadcasted_iota(jnp.int32, (KD, W4), 1) // KD
    blockmask = (mi == mj).astype(jnp.float32)
    wt_t = jnp.concatenate([wt, wt, wt, wt], axis=1)
    wa_t = jnp.concatenate([wa, wa, wa, wa], axis=1)
    w4t = blockmask * jnp.concatenate([wt_t] * 4, axis=0)
    w4a = blockmask * jnp.concatenate([wa_t] * 4, axis=0)
    bt4 = jnp.concatenate([bt_ref[...]] * 4, axis=1)
    ba4 = jnp.concatenate([ba_ref[...]] * 4, axis=1)
    t4 = jnp.dot(p, w4t, preferred_element_type=jnp.float32) + bt4
    a4 = jnp.dot(p, w4a, preferred_element_type=jnp.float32) + ba4
    ier4 = jnp.concatenate(
        [jnp.concatenate([g[:, s * EMB:(s + 1) * EMB]] * 4, axis=1)
         for s in range(4)], axis=1)
    q4 = t4 * ier4
    l4 = a4 * ier4
    ri = lax.broadcasted_iota(jnp.int32, (W4, W4), 0) // EMB
    ci = lax.broadcasted_iota(jnp.int32, (W4, W4), 1) // EMB
    s4 = (ri == ci).astype(jnp.float32)
    z4 = jnp.dot(l4, s4, preferred_element_type=jnp.float32)
    e4 = jnp.exp(z4)
    gi = lax.broadcasted_iota(jnp.int32, (W4, 4), 0) // KD
    gj = lax.broadcasted_iota(jnp.int32, (W4, 4), 1)
    gc = (gi == gj).astype(jnp.float32)
    denom = jnp.dot(e4, gc, preferred_element_type=jnp.float32)
    num = jnp.dot(e4 * q4, gc, preferred_element_type=jnp.float32)
    out_ref[...] = num * float(EMB) / denom + pub_ref[...] + pib_ref[...]


def _tc_mix(pue, pie, pub, pib, Wt, bt, Wa, ba):
    grid = (N4 // BLKN,)
    big = pl.BlockSpec((BLKN, KD), lambda i: (i, 0))
    small = pl.BlockSpec((BLKN, 4), lambda i: (i, 0))
    w = pl.BlockSpec((EMB, KD), lambda i: (0, 0))
    bias = pl.BlockSpec((1, KD), lambda i: (0, 0))
    return pl.pallas_call(
        _tc_mix_body,
        grid=grid,
        in_specs=[big, big, small, small, w, bias, w, bias],
        out_specs=small,
        out_shape=jax.ShapeDtypeStruct((N4, 4), jnp.float32),
    )(pue, pie, pub, pib, Wt, bt, Wa, ba)


@jax.jit
def kernel(user_ids, item_ids, user_emb, item_emb, user_bias, item_bias,
           Wt, bt, Wa, ba):
    uids = user_ids.astype(jnp.int32)
    iids = item_ids.astype(jnp.int32)
    ue, ie, ub, ib = _sc_gather(uids, iids, user_emb, item_emb,
                                user_bias, item_bias)
    pue = ue.reshape(N4, KD)
    pie = ie.reshape(N4, KD)
    pub = ub.reshape(N4, 4)
    pib = ib.reshape(N4, 4)
    out = _tc_mix(pue, pie, pub, pib,
                  Wt, bt.reshape(1, KD), Wa, ba.reshape(1, KD))
    return out.reshape(-1)


# speedup vs baseline: 1.1029x; 1.1029x over previous
import jax
import jax.numpy as jnp
from jax import lax
from jax.experimental import pallas as pl
from jax.experimental.pallas import tpu as pltpu
from jax.experimental.pallas import tpu_sc as plsc

B = 16384
EMB = 32
K = 4
KD = EMB * K
NROWS = 1000000
PACK = KD // EMB
BROWS = NROWS // KD + 1

NC = 2
NS = 16
NW = NC * NS
BPW = B // NW
IC = 128
NCHUNK = BPW // IC


def _sc_gather_body(uidq_hbm, iidq_hbm, uemb_hbm, iemb_hbm, ubias_hbm, ibias_hbm,
                    ue_out, ie_out, ub_out, ib_out,
                    uidq_v, iidq_v, bidx_v, rows_v, sem):
    wid = lax.axis_index("s") * NC + lax.axis_index("c")
    base = wid * BPW
    pltpu.sync_copy(uidq_hbm, uidq_v)
    pltpu.sync_copy(iidq_hbm, iidq_v)
    for tbl, idx_v, out in ((uemb_hbm, uidq_v, ue_out), (iemb_hbm, iidq_v, ie_out)):
        copies = []
        for j in range(NCHUNK):
            copies.append(pltpu.make_async_copy(
                tbl.at[idx_v.at[wid * NCHUNK + j]],
                rows_v.at[pl.ds(j * IC, IC)], sem))
        for c in copies:
            c.start()
        for c in copies:
            c.wait()
        pltpu.sync_copy(rows_v, out.at[pl.ds(base, BPW)])
    for idx_v, k, tbl, out in ((uidq_v, 0, ubias_hbm, ub_out),
                               (iidq_v, 1, ibias_hbm, ib_out)):
        for j in range(NCHUNK):
            for t in range(IC // 16):
                q = idx_v[wid * NCHUNK + j, pl.ds(t * 16, 16)]
                bidx_v[j, pl.ds(t * 16, 16)] = jnp.right_shift(q, 5)
        copies = []
        for j in range(NCHUNK):
            copies.append(pltpu.make_async_copy(
                tbl.at[bidx_v.at[j]],
                rows_v.at[pl.ds(j * IC, IC)], sem))
        for c in copies:
            c.start()
        for c in copies:
            c.wait()
        pltpu.sync_copy(rows_v, out.at[pl.ds(base, BPW)])


def _sc_gather(uidq, iidq, uemb128, iemb128, ubias128, ibias128):
    mesh = plsc.VectorSubcoreMesh(core_axis_name="c", subcore_axis_name="s",
                                  num_cores=NC, num_subcores=NS)
    f = pl.kernel(
        _sc_gather_body,
        out_type=(
            jax.ShapeDtypeStruct((B, KD), jnp.float32),
            jax.ShapeDtypeStruct((B, KD), jnp.float32),
            jax.ShapeDtypeStruct((B, KD), jnp.float32),
            jax.ShapeDtypeStruct((B, KD), jnp.float32),
        ),
        mesh=mesh,
        compiler_params=pltpu.CompilerParams(use_tc_tiling_on_sc=True),
        scratch_types=[
            pltpu.VMEM((B // IC, IC), jnp.int32),
            pltpu.VMEM((B // IC, IC), jnp.int32),
            pltpu.VMEM((NCHUNK, IC), jnp.int32),
            pltpu.VMEM((BPW, KD), jnp.float32),
            pltpu.SemaphoreType.DMA,
        ],
    )
    return f(uidq, iidq, uemb128, iemb128, ubias128, ibias128)


BLK = 2048


def _tc_mix_body(ue4_ref, ie4_ref, uo_ref, io_ref, ul_ref, il_ref,
                 ub4_ref, ib4_ref, wt_ref, bt_ref, wa_ref, ba_ref, out_ref):
    uo = uo_ref[...]
    io = io_ref[...]
    ue4 = ue4_ref[...]
    ie4 = ie4_ref[...]
    ue = jnp.zeros((BLK, EMB), jnp.float32)
    ie = jnp.zeros((BLK, EMB), jnp.float32)
    for o in range(PACK):
        ue = ue + jnp.where(uo == o, ue4[:, o * EMB:(o + 1) * EMB], 0.0)
        ie = ie + jnp.where(io == o, ie4[:, o * EMB:(o + 1) * EMB], 0.0)
    lane = lax.broadcasted_iota(jnp.int32, (BLK, KD), 1)
    ub = jnp.sum(jnp.where(lane == ul_ref[...], ub4_ref[...], 0.0),
                 axis=1, keepdims=True)
    ib = jnp.sum(jnp.where(lane == il_ref[...], ib4_ref[...], 0.0),
                 axis=1, keepdims=True)
    t = jnp.dot(ue, wt_ref[...], preferred_element_type=jnp.float32) + bt_ref[...]
    a = jnp.dot(ue, wa_ref[...], preferred_element_type=jnp.float32) + ba_ref[...]
    ier = jnp.concatenate([ie, ie, ie, ie], axis=1)
    q = t * ier
    l = a * ier
    ri = lax.broadcasted_iota(jnp.int32, (KD, KD), 0) // EMB
    ci = lax.broadcasted_iota(jnp.int32, (KD, KD), 1) // EMB
    s = (ri == ci).astype(jnp.float32)
    z = jnp.dot(l, s, preferred_element_type=jnp.float32)
    e = jnp.exp(z)
    denom = jnp.sum(e, axis=1, keepdims=True)
    num = jnp.sum(e * q, axis=1, keepdims=True)
    out_ref[...] = num * float(EMB) / denom + ub + ib


def _tc_mix(ue4, ie4, uo, io, ul, il, ub4, ib4, Wt, bt, Wa, ba):
    grid = (B // BLK,)
    big = pl.BlockSpec((BLK, KD), lambda i: (i, 0))
    col = pl.BlockSpec((BLK, 1), lambda i: (i, 0))
    w = pl.BlockSpec((EMB, KD), lambda i: (0, 0))
    bias = pl.BlockSpec((1, KD), lambda i: (0, 0))
    return pl.pallas_call(
        _tc_mix_body,
        grid=grid,
        in_specs=[big, big, col, col, col, col, big, big, w, bias, w, bias],
        out_specs=col,
        out_shape=jax.ShapeDtypeStruct((B, 1), jnp.float32),
    )(ue4, ie4, uo, io, ul, il, ub4, ib4, Wt, bt, Wa, ba)


@jax.jit
def kernel(user_ids, item_ids, user_emb, item_emb, user_bias, item_bias,
           Wt, bt, Wa, ba):
    uids = user_ids.astype(jnp.int32)
    iids = item_ids.astype(jnp.int32)
    uidq = (uids // PACK).reshape(B // IC, IC)
    iidq = (iids // PACK).reshape(B // IC, IC)
    uemb128 = user_emb.reshape(NROWS * EMB // KD, KD)
    iemb128 = item_emb.reshape(NROWS * EMB // KD, KD)
    ubias128 = jnp.pad(user_bias.reshape(-1), (0, BROWS * KD - NROWS)).reshape(BROWS, KD)
    ibias128 = jnp.pad(item_bias.reshape(-1), (0, BROWS * KD - NROWS)).reshape(BROWS, KD)
    ue4, ie4, ub4, ib4 = _sc_gather(uidq, iidq, uemb128, iemb128,
                                    ubias128, ibias128)
    uo = (uids % PACK).reshape(B, 1)
    io = (iids % PACK).reshape(B, 1)
    ul = (uids % KD).reshape(B, 1)
    il = (iids % KD).reshape(B, 1)
    out = _tc_mix(ue4, ie4, uo, io, ul, il, ub4, ib4,
                  Wt, bt.reshape(1, KD), Wa, ba.reshape(1, KD))
    return out.reshape(-1)
